# baseline (speedup 1.0000x reference)
import functools

import jax
import jax.numpy as jnp
from jax import lax
from jax.experimental import pallas as pl
from jax.experimental.pallas import tpu as pltpu
from jax.experimental.pallas import tpu_sc as plsc

N_CLASSES = 1000
CHUNK = 32


@functools.partial(jax.jit, static_argnums=(1, 2))
def _one_hot_sc(xi, b, n):
    info = plsc.get_sparse_core_info()
    nc, ns, lanes = info.num_cores, info.num_subcores, info.num_lanes
    nw = nc * ns
    assert b % (nw * CHUNK) == 0
    b_per_w = b // nw
    n_chunks = b_per_w // CHUNK
    n_pairs = n_chunks // 2
    vecs_per_chunk = CHUNK // lanes

    mesh = plsc.VectorSubcoreMesh(core_axis_name="c", subcore_axis_name="s")

    @functools.partial(
        pl.kernel,
        mesh=mesh,
        out_type=jax.ShapeDtypeStruct((b * n,), jnp.float32),
        compiler_params=pltpu.CompilerParams(needs_layout_passes=False),
        scratch_types=[
            pltpu.VMEM((b_per_w,), jnp.int32),
            pltpu.VMEM((CHUNK * n,), jnp.float32),
            pltpu.VMEM((CHUNK * n,), jnp.float32),
            pltpu.SemaphoreType.DMA,
            pltpu.SemaphoreType.DMA,
        ],
    )
    def k(x_hbm, out_hbm, idx_v, buf0, buf1, sem0, sem1):
        bufs = (buf0, buf1)
        sems = (sem0, sem1)
        wid = lax.axis_index("s") * nc + lax.axis_index("c")
        base = wid * b_per_w
        pltpu.sync_copy(x_hbm.at[pl.ds(base, b_per_w)], idx_v)

        zeros = jnp.zeros((lanes,), jnp.float32)
        ones = jnp.full((lanes,), 1.0, jnp.float32)
        row_off = lax.iota(jnp.int32, lanes) * n

        def zero_fill(i, _):
            buf0[pl.ds(i * lanes, lanes)] = zeros
            buf1[pl.ds(i * lanes, lanes)] = zeros
            return 0

        lax.fori_loop(0, CHUNK * n // lanes, zero_fill, 0)

        def flat_pos(g, j):
            v = idx_v[pl.ds(g * CHUNK + j * lanes, lanes)]
            return row_off + (j * lanes * n) + v

        def set_ones_and_send(g, slot):
            for j in range(vecs_per_chunk):
                plsc.store_scatter(bufs[slot], [flat_pos(g, j)], ones)
            dst = out_hbm.at[pl.ds((base + g * CHUNK) * n, CHUNK * n)]
            pltpu.make_async_copy(bufs[slot], dst, sems[slot]).start()

        set_ones_and_send(0, 0)
        set_ones_and_send(1, 1)

        def pair_body(p, _):
            for slot in range(2):
                g = 2 * p + slot
                dst = out_hbm.at[pl.ds((base + g * CHUNK) * n, CHUNK * n)]
                pltpu.make_async_copy(bufs[slot], dst, sems[slot]).wait()
                for j in range(vecs_per_chunk):
                    plsc.store_scatter(bufs[slot], [flat_pos(g - 2, j)], zeros)
                set_ones_and_send(g, slot)
            return 0

        lax.fori_loop(1, n_pairs, pair_body, 0)

        for slot in range(2):
            g = n_chunks - 2 + slot
            dst = out_hbm.at[pl.ds((base + g * CHUNK) * n, CHUNK * n)]
            pltpu.make_async_copy(bufs[slot], dst, sems[slot]).wait()

    return k(xi)


def kernel(x):
    b0, b1 = x.shape
    xi = x.reshape(b0 * b1).astype(jnp.int32)
    out = _one_hot_sc(xi, b0 * b1, N_CLASSES)
    return out.reshape(b0, b1, N_CLASSES)
